# baseline (speedup 1.0000x reference)
import functools
import jax
import jax.numpy as jnp
from jax.experimental import pallas as pl
from jax.experimental.pallas import tpu as pltpu

_N = 100000
_BLK = 4000


def _sage_dense_body(summed_ref, cnt_ref, x_ref, wl_ref, bl_ref, wr_ref, o_ref):
    cnt = jnp.maximum(cnt_ref[...], 1.0)
    mean = summed_ref[...] / cnt
    h = (mean @ wl_ref[...] + bl_ref[...][None, :]
         + x_ref[...] @ wr_ref[...])
    o_ref[...] = jnp.maximum(h, 0.0)


def _sage_dense(summed, cnt, x, wl, bl, wr):
    n, d = x.shape
    grid = n // _BLK
    return pl.pallas_call(
        _sage_dense_body,
        grid=(grid,),
        in_specs=[
            pl.BlockSpec((_BLK, d), lambda i: (i, 0)),
            pl.BlockSpec((_BLK, 1), lambda i: (i, 0)),
            pl.BlockSpec((_BLK, d), lambda i: (i, 0)),
            pl.BlockSpec((d, d), lambda i: (0, 0)),
            pl.BlockSpec((d,), lambda i: (0,)),
            pl.BlockSpec((d, d), lambda i: (0, 0)),
        ],
        out_specs=pl.BlockSpec((_BLK, d), lambda i: (i, 0)),
        out_shape=jax.ShapeDtypeStruct((n, d), x.dtype),
    )(summed, cnt, x, wl, bl, wr)


def _fc_body(h_ref, w_ref, b_ref, o_ref):
    o_ref[...] = h_ref[...] @ w_ref[...] + b_ref[...][None, :]


def _fc(h, w, b):
    n, d = h.shape
    do = w.shape[1]
    return pl.pallas_call(
        _fc_body,
        grid=(n // _BLK,),
        in_specs=[
            pl.BlockSpec((_BLK, d), lambda i: (i, 0)),
            pl.BlockSpec((d, do), lambda i: (0, 0)),
            pl.BlockSpec((do,), lambda i: (0,)),
        ],
        out_specs=pl.BlockSpec((_BLK, do), lambda i: (i, 0)),
        out_shape=jax.ShapeDtypeStruct((n, do), h.dtype),
    )(h, w, b)


def kernel(x, edge_index, W1_l, b1_l, W1_r, W2_l, b2_l, W2_r, Wfc, bfc):
    src = edge_index[0].astype(jnp.int32)
    dst = edge_index[1].astype(jnp.int32)
    n = x.shape[0]
    summed1 = jnp.zeros_like(x).at[dst].add(jnp.take(x, src, axis=0))
    cnt = jnp.zeros((n, 1), x.dtype).at[dst, 0].add(1.0)
    h1 = _sage_dense(summed1, cnt, x, W1_l, b1_l, W1_r)
    summed2 = jnp.zeros_like(h1).at[dst].add(jnp.take(h1, src, axis=0))
    h2 = _sage_dense(summed2, cnt, h1, W2_l, b2_l, W2_r)
    out = _fc(h2, Wfc, bfc)
    return (out, h2)
